# Initial kernel scaffold; baseline (speedup 1.0000x reference)
import jax
import jax.numpy as jnp
from jax.experimental import pallas as pl


def kernel(x, codebook):
    raise NotImplementedError("write your pallas kernel here")


# speedup vs baseline: 1.4640x; 1.4640x over previous
import functools

import jax
import jax.numpy as jnp
from jax import lax
from jax.experimental import pallas as pl
from jax.experimental.pallas import tpu as pltpu
from jax.experimental.pallas import tpu_sc as plsc


_BLK = 512


def _assign_body(x_ref, cbt_ref, idx_ref):
    xb = x_ref[...]
    cbt = cbt_ref[...]
    c_sq = jnp.sum(cbt * cbt, axis=0, keepdims=True)
    dots = lax.dot_general(
        xb, cbt, (((1,), (0,)), ((), ())),
        preferred_element_type=jnp.float32)
    scores = c_sq - 2.0 * dots
    m = jnp.min(scores, axis=1, keepdims=True)
    k = scores.shape[1]
    col = lax.broadcasted_iota(jnp.int32, scores.shape, 1)
    idx = jnp.min(jnp.where(scores <= m, col, k), axis=1)
    idx_ref[0, 0, :] = idx


def _assign(flat, cbt):
    n, c = flat.shape
    k = cbt.shape[1]
    nb = n // _BLK
    idx3 = pl.pallas_call(
        _assign_body,
        grid=(nb,),
        in_specs=[
            pl.BlockSpec((_BLK, c), lambda i: (i, 0)),
            pl.BlockSpec((c, k), lambda i: (0, 0)),
        ],
        out_specs=pl.BlockSpec((1, 1, _BLK), lambda i: (i, 0, 0)),
        out_shape=jax.ShapeDtypeStruct((nb, 1, _BLK), jnp.int32),
    )(flat, cbt)
    return idx3.reshape(n)


@functools.lru_cache(maxsize=None)
def _make_gather(v, d, b):
    info = plsc.get_sparse_core_info()
    nc, ns = info.num_cores, info.num_subcores
    nw = nc * ns
    assert b % (8 * nw) == 0 and d % info.num_lanes == 0
    b_per_w = b // nw
    mesh = plsc.VectorSubcoreMesh(core_axis_name="c", subcore_axis_name="s")

    @functools.partial(
        pl.kernel,
        mesh=mesh,
        out_type=jax.ShapeDtypeStruct((b, d), jnp.float32),
        scratch_types=[
            pltpu.VMEM((b_per_w,), jnp.int32),
            pltpu.VMEM((b_per_w, d), jnp.float32),
            pltpu.SemaphoreType.DMA,
        ],
        compiler_params=pltpu.CompilerParams(use_tc_tiling_on_sc=False),
    )
    def gather(table_hbm, idx_hbm, out_hbm, idx_v, rows_v, sem):
        wid = lax.axis_index("s") * nc + lax.axis_index("c")
        base = wid * b_per_w
        pltpu.sync_copy(idx_hbm.at[pl.ds(base, b_per_w)], idx_v)
        pltpu.async_copy(table_hbm.at[idx_v], rows_v, sem).wait()
        pltpu.sync_copy(rows_v, out_hbm.at[pl.ds(base, b_per_w)])

    return gather


def kernel(x, codebook):
    b, h, w, c = x.shape
    n = b * h * w
    flat = x.reshape(n, c)
    idx = _assign(flat, codebook.T)
    flat_y = _make_gather(codebook.shape[0], c, n)(codebook, idx)
    y = flat_y.reshape(b, h, w, c)
    return (x, y)


# speedup vs baseline: 2.7753x; 1.8957x over previous
import functools

import jax
import jax.numpy as jnp
from jax import lax
from jax.experimental import pallas as pl
from jax.experimental.pallas import tpu as pltpu
from jax.experimental.pallas import tpu_sc as plsc


_BLK = 14336


_CA = 40


def _assign_body(x_ref, cb_ref, idx_ref, cba_ref, xba_ref):
    c = x_ref.shape[0]
    k = cb_ref.shape[0]
    blk = x_ref.shape[1]

    @pl.when(pl.program_id(0) == 0)
    def _prep():
        cb = cb_ref[...]
        c_sq = jnp.sum(cb * cb, axis=1, keepdims=True)
        hi = c_sq.astype(jnp.bfloat16).astype(jnp.float32)
        mid = (c_sq - hi).astype(jnp.bfloat16).astype(jnp.float32)
        lo = (c_sq - hi - mid).astype(jnp.bfloat16).astype(jnp.float32)
        cba_ref[:, 0:c] = -2.0 * cb
        cba_ref[:, c:c + 1] = hi
        cba_ref[:, c + 1:c + 2] = mid
        cba_ref[:, c + 2:c + 3] = lo
        cba_ref[:, c + 3:] = jnp.zeros((k, _CA - c - 3), jnp.float32)
        xba_ref[c:c + 3, :] = jnp.ones((3, blk), jnp.float32)
        xba_ref[c + 3:, :] = jnp.zeros((_CA - c - 3, blk), jnp.float32)

    xba_ref[0:c, :] = x_ref[...]
    scores = lax.dot_general(
        cba_ref[...], xba_ref[...], (((1,), (0,)), ((), ())),
        preferred_element_type=jnp.float32)
    idx = jnp.argmin(scores, axis=0).astype(jnp.int32)
    idx_ref[0, 0, :] = idx


def _assign(flat_t, codebook):
    c, n = flat_t.shape
    k = codebook.shape[0]
    nb = n // _BLK
    idx3 = pl.pallas_call(
        _assign_body,
        grid=(nb,),
        in_specs=[
            pl.BlockSpec((c, _BLK), lambda i: (0, i)),
            pl.BlockSpec((k, c), lambda i: (0, 0)),
        ],
        out_specs=pl.BlockSpec((1, 1, _BLK), lambda i: (i, 0, 0)),
        out_shape=jax.ShapeDtypeStruct((nb, 1, _BLK), jnp.int32),
        scratch_shapes=[
            pltpu.VMEM((k, _CA), jnp.float32),
            pltpu.VMEM((_CA, _BLK), jnp.float32),
        ],
    )(flat_t, codebook)
    return idx3.reshape(n)


@functools.lru_cache(maxsize=None)
def _make_gather(v, d, b):
    info = plsc.get_sparse_core_info()
    nc, ns = info.num_cores, info.num_subcores
    nw = nc * ns
    assert b % (8 * nw) == 0 and d % info.num_lanes == 0
    b_per_w = b // nw
    mesh = plsc.VectorSubcoreMesh(core_axis_name="c", subcore_axis_name="s")

    @functools.partial(
        pl.kernel,
        mesh=mesh,
        out_type=jax.ShapeDtypeStruct((b, d), jnp.float32),
        scratch_types=[
            pltpu.VMEM((b_per_w,), jnp.int32),
            pltpu.VMEM((b_per_w, d), jnp.float32),
            pltpu.SemaphoreType.DMA,
        ],
        compiler_params=pltpu.CompilerParams(use_tc_tiling_on_sc=False),
    )
    def gather(table_hbm, idx_hbm, out_hbm, idx_v, rows_v, sem):
        wid = lax.axis_index("s") * nc + lax.axis_index("c")
        base = wid * b_per_w
        pltpu.sync_copy(idx_hbm.at[pl.ds(base, b_per_w)], idx_v)
        pltpu.async_copy(table_hbm.at[idx_v], rows_v, sem).wait()
        pltpu.sync_copy(rows_v, out_hbm.at[pl.ds(base, b_per_w)])

    return gather


def kernel(x, codebook):
    b, h, w, c = x.shape
    n = b * h * w
    flat = x.reshape(n, c)
    idx = _assign(flat.T, codebook)
    flat_y = _make_gather(codebook.shape[0], c, n)(codebook, idx)
    y = flat_y.reshape(b, h, w, c)
    return (x, y)


# speedup vs baseline: 2.9519x; 1.0636x over previous
import functools

import jax
import jax.numpy as jnp
from jax import lax
from jax.experimental import pallas as pl
from jax.experimental.pallas import tpu as pltpu
from jax.experimental.pallas import tpu_sc as plsc


_BLK = 14336


_CA = 40


def _assign_body(x_ref, cb_ref, idx_ref, xout_ref, cba_ref, xba_ref):
    c = x_ref.shape[1]
    k = cb_ref.shape[0]
    blk = x_ref.shape[0]

    @pl.when(pl.program_id(0) == 0)
    def _prep():
        cb = cb_ref[...]
        c_sq = jnp.sum(cb * cb, axis=1, keepdims=True)
        hi = c_sq.astype(jnp.bfloat16).astype(jnp.float32)
        mid = (c_sq - hi).astype(jnp.bfloat16).astype(jnp.float32)
        lo = (c_sq - hi - mid).astype(jnp.bfloat16).astype(jnp.float32)
        cba_ref[:, 0:c] = -2.0 * cb
        cba_ref[:, c:c + 1] = hi
        cba_ref[:, c + 1:c + 2] = mid
        cba_ref[:, c + 2:c + 3] = lo
        cba_ref[:, c + 3:] = jnp.zeros((k, _CA - c - 3), jnp.float32)
        xba_ref[c:c + 3, :] = jnp.ones((3, blk), jnp.float32)
        xba_ref[c + 3:, :] = jnp.zeros((_CA - c - 3, blk), jnp.float32)

    xb = x_ref[...]
    xout_ref[...] = xb
    xba_ref[0:c, :] = xb.T
    scores = lax.dot_general(
        cba_ref[...], xba_ref[...], (((1,), (0,)), ((), ())),
        preferred_element_type=jnp.float32)
    idx = jnp.argmin(scores, axis=0).astype(jnp.int32)
    idx_ref[0, 0, :] = idx


def _assign(flat, codebook):
    n, c = flat.shape
    k = codebook.shape[0]
    nb = n // _BLK
    idx3, xout = pl.pallas_call(
        _assign_body,
        grid=(nb,),
        in_specs=[
            pl.BlockSpec((_BLK, c), lambda i: (i, 0)),
            pl.BlockSpec((k, c), lambda i: (0, 0)),
        ],
        out_specs=[
            pl.BlockSpec((1, 1, _BLK), lambda i: (i, 0, 0)),
            pl.BlockSpec((_BLK, c), lambda i: (i, 0)),
        ],
        out_shape=[
            jax.ShapeDtypeStruct((nb, 1, _BLK), jnp.int32),
            jax.ShapeDtypeStruct((n, c), jnp.float32),
        ],
        scratch_shapes=[
            pltpu.VMEM((k, _CA), jnp.float32),
            pltpu.VMEM((_CA, _BLK), jnp.float32),
        ],
    )(flat, codebook)
    return idx3.reshape(n), xout


@functools.lru_cache(maxsize=None)
def _make_gather(v, d, b):
    info = plsc.get_sparse_core_info()
    nc, ns = info.num_cores, info.num_subcores
    nw = nc * ns
    assert b % (8 * nw) == 0 and d % info.num_lanes == 0
    b_per_w = b // nw
    mesh = plsc.VectorSubcoreMesh(core_axis_name="c", subcore_axis_name="s")

    @functools.partial(
        pl.kernel,
        mesh=mesh,
        out_type=jax.ShapeDtypeStruct((b, d), jnp.float32),
        scratch_types=[
            pltpu.VMEM((b_per_w,), jnp.int32),
            pltpu.VMEM((b_per_w, d), jnp.float32),
            pltpu.SemaphoreType.DMA,
        ],
        compiler_params=pltpu.CompilerParams(use_tc_tiling_on_sc=False),
    )
    def gather(table_hbm, idx_hbm, out_hbm, idx_v, rows_v, sem):
        wid = lax.axis_index("s") * nc + lax.axis_index("c")
        base = wid * b_per_w
        pltpu.sync_copy(idx_hbm.at[pl.ds(base, b_per_w)], idx_v)
        pltpu.async_copy(table_hbm.at[idx_v], rows_v, sem).wait()
        pltpu.sync_copy(rows_v, out_hbm.at[pl.ds(base, b_per_w)])

    return gather


def kernel(x, codebook):
    b, h, w, c = x.shape
    n = b * h * w
    flat = x.reshape(n, c)
    idx, xout = _assign(flat, codebook)
    flat_y = _make_gather(codebook.shape[0], c, n)(codebook, idx)
    y = flat_y.reshape(b, h, w, c)
    return (xout.reshape(b, h, w, c), y)


# speedup vs baseline: 3.5366x; 1.1981x over previous
import functools

import jax
import jax.numpy as jnp
from jax import lax
from jax.experimental import pallas as pl
from jax.experimental.pallas import tpu as pltpu
from jax.experimental.pallas import tpu_sc as plsc


_G = 64


_CA = 40


def _assign_body(x_ref, cb_ref, idx_ref, xout_ref, cba_ref, xba_ref):
    g, c, w = x_ref.shape
    k = cb_ref.shape[0]
    blk = g * w

    @pl.when(pl.program_id(0) == 0)
    def _prep():
        cb = cb_ref[...]
        c_sq = jnp.sum(cb * cb, axis=1, keepdims=True)
        hi = c_sq.astype(jnp.bfloat16).astype(jnp.float32)
        mid = (c_sq - hi).astype(jnp.bfloat16).astype(jnp.float32)
        lo = (c_sq - hi - mid).astype(jnp.bfloat16).astype(jnp.float32)
        cba_ref[:, 0:c] = -2.0 * cb
        cba_ref[:, c:c + 1] = hi
        cba_ref[:, c + 1:c + 2] = mid
        cba_ref[:, c + 2:c + 3] = lo
        cba_ref[:, c + 3:] = jnp.zeros((k, _CA - c - 3), jnp.float32)
        xba_ref[c:c + 3, :] = jnp.ones((3, blk), jnp.float32)
        xba_ref[c + 3:, :] = jnp.zeros((_CA - c - 3, blk), jnp.float32)

    xb3 = x_ref[...]
    xout_ref[...] = xb3
    xba_ref[0:c, :] = xb3.transpose(1, 0, 2).reshape(c, blk)
    scores = lax.dot_general(
        cba_ref[...], xba_ref[...], (((1,), (0,)), ((), ())),
        preferred_element_type=jnp.float32)
    idx = jnp.argmin(scores, axis=0).astype(jnp.int32)
    idx_ref[0, 0, :] = idx


def _assign(x3, codebook):
    gh, c, w = x3.shape
    n = gh * w
    k = codebook.shape[0]
    nb = gh // _G
    blk = _G * w
    idx3, xout3 = pl.pallas_call(
        _assign_body,
        grid=(nb,),
        in_specs=[
            pl.BlockSpec((_G, c, w), lambda i: (i, 0, 0)),
            pl.BlockSpec((k, c), lambda i: (0, 0)),
        ],
        out_specs=[
            pl.BlockSpec((1, 1, blk), lambda i: (i, 0, 0)),
            pl.BlockSpec((_G, c, w), lambda i: (i, 0, 0)),
        ],
        out_shape=[
            jax.ShapeDtypeStruct((nb, 1, blk), jnp.int32),
            jax.ShapeDtypeStruct((gh, c, w), jnp.float32),
        ],
        scratch_shapes=[
            pltpu.VMEM((k, _CA), jnp.float32),
            pltpu.VMEM((_CA, blk), jnp.float32),
        ],
    )(x3, codebook)
    return idx3.reshape(n), xout3


@functools.lru_cache(maxsize=None)
def _make_gather(v, d, b):
    info = plsc.get_sparse_core_info()
    nc, ns = info.num_cores, info.num_subcores
    nw = nc * ns
    assert b % (8 * nw) == 0 and d % info.num_lanes == 0
    b_per_w = b // nw
    mesh = plsc.VectorSubcoreMesh(core_axis_name="c", subcore_axis_name="s")

    @functools.partial(
        pl.kernel,
        mesh=mesh,
        out_type=jax.ShapeDtypeStruct((b, d), jnp.float32),
        scratch_types=[
            pltpu.VMEM((b_per_w,), jnp.int32),
            pltpu.VMEM((b_per_w, d), jnp.float32),
            pltpu.SemaphoreType.DMA,
        ],
        compiler_params=pltpu.CompilerParams(use_tc_tiling_on_sc=False),
    )
    def gather(table_hbm, idx_hbm, out_hbm, idx_v, rows_v, sem):
        wid = lax.axis_index("s") * nc + lax.axis_index("c")
        base = wid * b_per_w
        pltpu.sync_copy(idx_hbm.at[pl.ds(base, b_per_w)], idx_v)
        pltpu.async_copy(table_hbm.at[idx_v], rows_v, sem).wait()
        pltpu.sync_copy(rows_v, out_hbm.at[pl.ds(base, b_per_w)])

    return gather


def kernel(x, codebook):
    b, h, w, c = x.shape
    n = b * h * w
    x3 = jnp.transpose(x, (0, 1, 3, 2)).reshape(b * h, c, w)
    idx, xout3 = _assign(x3, codebook)
    flat_y = _make_gather(codebook.shape[0], c, n)(codebook, idx)
    y = flat_y.reshape(b, h, w, c)
    xout = jnp.transpose(xout3.reshape(b, h, c, w), (0, 1, 3, 2))
    return (xout, y)
